# baseline (speedup 1.0000x reference)
import math

import jax
import jax.numpy as jnp
from jax import lax
from jax.experimental import pallas as pl
from jax.experimental.pallas import tpu as pltpu

_B, _NF, _D, _FF, _E, _H = 1024, 20, 256, 512, 5, 8
_S = _NF + 1
_T = _B * _S
_DH = _D // _H
_TILE = 512
_NT = _T // _TILE
_G1 = 8
_G2 = 16
_PREC = lax.Precision.HIGHEST
_NEG = -1e30


def _f32dot(a, b):
    return jnp.dot(a, b, preferred_element_type=jnp.float32, precision=_PREC)


def _ln(x, g, b):
    m = jnp.mean(x, axis=-1, keepdims=True)
    v = jnp.mean((x - m) ** 2, axis=-1, keepdims=True)
    return (x - m) / jnp.sqrt(v + 1e-5) * g + b


def _full_spec(shape):
    n = len(shape)
    return pl.BlockSpec(shape, lambda i, _n=n: (0,) * _n)


def _div21(x):
    return (x * 3121) >> 16


def _embed_body(x_ref, w_ref, b_ref, cls_ref, g_ref, bb_ref, o_ref):
    x = x_ref[...]
    w = w_ref[...][None]
    b = b_ref[...][None]
    feat = x[:, :, None] * w + b
    cls = jnp.broadcast_to(cls_ref[...][None], (x.shape[0], 1, _D))
    h = jnp.concatenate([cls, feat], axis=1)
    o_ref[...] = _ln(h, g_ref[...][None], bb_ref[...][None])


def _embed(x, p):
    bb = 256
    return pl.pallas_call(
        _embed_body,
        grid=(_B // bb,),
        in_specs=[
            pl.BlockSpec((bb, _NF), lambda i: (i, 0)),
            _full_spec((1, _D)), _full_spec((1, _D)), _full_spec((1, _D)),
            _full_spec((1, _D)), _full_spec((1, _D)),
        ],
        out_specs=pl.BlockSpec((bb, _S, _D), lambda i: (i, 0, 0)),
        out_shape=jax.ShapeDtypeStruct((_B, _S, _D), jnp.float32),
    )(x, p['con_w'].reshape(1, _D), p['con_b'].reshape(1, _D),
      p['cls'].reshape(1, _D), p['norm_g'].reshape(1, _D),
      p['norm_b'].reshape(1, _D))


def _qkv_body(h_ref, g_ref, b_ref, w_ref, bias_ref, o_ref):
    u = _ln(h_ref[...], g_ref[...], b_ref[...])
    o_ref[...] = _f32dot(u, w_ref[...]) + bias_ref[...]


def _qkv(h, bp):
    return pl.pallas_call(
        _qkv_body,
        grid=(_NT,),
        in_specs=[
            pl.BlockSpec((_TILE, _D), lambda i: (i, 0)),
            _full_spec((1, _D)), _full_spec((1, _D)),
            _full_spec((_D, 3 * _D)), _full_spec((1, 3 * _D)),
        ],
        out_specs=pl.BlockSpec((_TILE, 3 * _D), lambda i: (i, 0)),
        out_shape=jax.ShapeDtypeStruct((_T, 3 * _D), jnp.float32),
    )(h, bp['ln1_g'].reshape(1, _D), bp['ln1_b'].reshape(1, _D),
      bp['wqkv'], bp['bqkv'].reshape(1, 3 * _D))


def _split_heads(t):
    return jnp.concatenate([t[:, :, _DH * h:_DH * (h + 1)] for h in range(_H)],
                           axis=1)


def _attn1_body(qkv_ref, o_ref):
    u = qkv_ref[...]
    qh = _split_heads(u[:, :, :_D])
    kh = _split_heads(u[:, :, _D:2 * _D])
    vh = _split_heads(u[:, :, 2 * _D:])
    s = lax.dot_general(qh, kh, (((2,), (2,)), ((0,), (0,))),
                        preferred_element_type=jnp.float32, precision=_PREC)
    hs = _H * _S
    rh = _div21(lax.broadcasted_iota(jnp.int32, (hs, hs), 0))
    ch = _div21(lax.broadcasted_iota(jnp.int32, (hs, hs), 1))
    s = jnp.where((rh == ch)[None], s * (1.0 / math.sqrt(_DH)), _NEG)
    s = s - jnp.max(s, axis=-1, keepdims=True)
    e = jnp.exp(s)
    a = e / jnp.sum(e, axis=-1, keepdims=True)
    o = lax.dot_general(a, vh, (((2,), (1,)), ((0,), (0,))),
                        preferred_element_type=jnp.float32, precision=_PREC)
    o_ref[...] = jnp.concatenate(
        [o[:, _S * h:_S * (h + 1), :] for h in range(_H)], axis=2)


def _attn1(qkv):
    return pl.pallas_call(
        _attn1_body,
        grid=(_B // _G1,),
        in_specs=[pl.BlockSpec((_G1, _S, 3 * _D), lambda i: (i, 0, 0))],
        out_specs=pl.BlockSpec((_G1, _S, _D), lambda i: (i, 0, 0)),
        out_shape=jax.ShapeDtypeStruct((_B, _S, _D), jnp.float32),
    )(qkv)


def _attn2_body(qkv_ref, o_ref):
    u = qkv_ref[...]
    q0 = u[:, 0:1, :_D]
    qh = _split_heads(q0)
    kh = _split_heads(u[:, :, _D:2 * _D])
    vh = _split_heads(u[:, :, 2 * _D:])
    s = lax.dot_general(qh, kh, (((2,), (2,)), ((0,), (0,))),
                        preferred_element_type=jnp.float32, precision=_PREC)
    hs = _H * _S
    rh = lax.broadcasted_iota(jnp.int32, (_H, hs), 0)
    ch = _div21(lax.broadcasted_iota(jnp.int32, (_H, hs), 1))
    s = jnp.where((rh == ch)[None], s * (1.0 / math.sqrt(_DH)), _NEG)
    s = s - jnp.max(s, axis=-1, keepdims=True)
    e = jnp.exp(s)
    a = e / jnp.sum(e, axis=-1, keepdims=True)
    o = lax.dot_general(a, vh, (((2,), (1,)), ((0,), (0,))),
                        preferred_element_type=jnp.float32, precision=_PREC)
    o_ref[...] = jnp.concatenate(
        [o[:, h:h + 1, :] for h in range(_H)], axis=2)


def _attn2(qkv):
    return pl.pallas_call(
        _attn2_body,
        grid=(_B // _G2,),
        in_specs=[pl.BlockSpec((_G2, _S, 3 * _D), lambda i: (i, 0, 0))],
        out_specs=pl.BlockSpec((_G2, 1, _D), lambda i: (i, 0, 0)),
        out_shape=jax.ShapeDtypeStruct((_B, 1, _D), jnp.float32),
    )(qkv)


def _proj_body(h_ref, a_ref, w_ref, b_ref, o_ref):
    o_ref[...] = h_ref[...] + _f32dot(a_ref[...], w_ref[...]) + b_ref[...]


def _proj(h, att, bp):
    return pl.pallas_call(
        _proj_body,
        grid=(_NT,),
        in_specs=[
            pl.BlockSpec((_TILE, _D), lambda i: (i, 0)),
            pl.BlockSpec((_TILE, _D), lambda i: (i, 0)),
            _full_spec((_D, _D)), _full_spec((1, _D)),
        ],
        out_specs=pl.BlockSpec((_TILE, _D), lambda i: (i, 0)),
        out_shape=jax.ShapeDtypeStruct((_T, _D), jnp.float32),
    )(h, att, bp['wo'], bp['bo'].reshape(1, _D))


def _moe_math(h, g2, b2, cent, w1, b1, w2, b2e, ws1, bs1, ws2, bs2):
    u2 = _ln(h, g2, b2)
    logits = _f32dot(u2, cent)
    mx = jnp.max(logits, axis=-1, keepdims=True)
    eg = jnp.exp(logits - mx)
    gate = eg / jnp.sum(eg, axis=-1, keepdims=True)
    iot = lax.broadcasted_iota(jnp.int32, logits.shape, 1)
    top = jnp.min(jnp.where(logits == mx, iot, _E), axis=-1, keepdims=True)
    acc = jnp.zeros_like(h)
    for e in range(_E):
        t1 = jax.nn.gelu(_f32dot(u2, w1[e]) + b1[e][None])
        t2 = _f32dot(t1, w2[e]) + b2e[e][None]
        sel = jnp.where(top == e, gate[:, e:e + 1], 0.0)
        acc = acc + sel * t2
    ys = _f32dot(jax.nn.gelu(_f32dot(u2, ws1) + bs1), ws2) + bs2
    return h + acc + ys


def _moe_body(h_ref, g2_ref, b2_ref, cent_ref, w1_ref, b1_ref, w2_ref,
              b2e_ref, ws1_ref, bs1_ref, ws2_ref, bs2_ref, o_ref):
    o_ref[...] = _moe_math(
        h_ref[...], g2_ref[...], b2_ref[...], cent_ref[...], w1_ref[...],
        b1_ref[...], w2_ref[...], b2e_ref[...], ws1_ref[...], bs1_ref[...],
        ws2_ref[...], bs2_ref[...])


def _moe_dense(h, bp, cent_t):
    return pl.pallas_call(
        _moe_body,
        grid=(_NT,),
        in_specs=[
            pl.BlockSpec((_TILE, _D), lambda i: (i, 0)),
            _full_spec((1, _D)), _full_spec((1, _D)), _full_spec((_D, _E)),
            _full_spec((_E, _D, _FF)), _full_spec((_E, _FF)),
            _full_spec((_E, _FF, _D)), _full_spec((_E, _D)),
            _full_spec((_D, _FF)), _full_spec((1, _FF)),
            _full_spec((_FF, _D)), _full_spec((1, _D)),
        ],
        out_specs=pl.BlockSpec((_TILE, _D), lambda i: (i, 0)),
        out_shape=jax.ShapeDtypeStruct((_T, _D), jnp.float32),
    )(h, bp['ln2_g'].reshape(1, _D), bp['ln2_b'].reshape(1, _D), cent_t,
      bp['w1'], bp['b1'], bp['w2'], bp['b2'],
      bp['ws1'], bp['bs1'].reshape(1, _FF), bp['ws2'],
      bp['bs2'].reshape(1, _D))


def _final_body(hc_ref, a2_ref, wo_ref, bo_ref, g2_ref, b2_ref, cent_ref,
                w1_ref, b1_ref, w2_ref, b2e_ref, ws1_ref, bs1_ref, ws2_ref,
                bs2_ref, fcw_ref, fcb_ref, o_ref):
    c = hc_ref[...] + _f32dot(a2_ref[...], wo_ref[...]) + bo_ref[...]
    o = _moe_math(c, g2_ref[...], b2_ref[...], cent_ref[...], w1_ref[...],
                  b1_ref[...], w2_ref[...], b2e_ref[...], ws1_ref[...],
                  bs1_ref[...], ws2_ref[...], bs2_ref[...])
    o_ref[...] = _f32dot(o, fcw_ref[...]) + fcb_ref[...]


def _final(hcls, att2, bp, cent_t, p):
    bb = 512
    return pl.pallas_call(
        _final_body,
        grid=(_B // bb,),
        in_specs=[
            pl.BlockSpec((bb, _D), lambda i: (i, 0)),
            pl.BlockSpec((bb, _D), lambda i: (i, 0)),
            _full_spec((_D, _D)), _full_spec((1, _D)),
            _full_spec((1, _D)), _full_spec((1, _D)), _full_spec((_D, _E)),
            _full_spec((_E, _D, _FF)), _full_spec((_E, _FF)),
            _full_spec((_E, _FF, _D)), _full_spec((_E, _D)),
            _full_spec((_D, _FF)), _full_spec((1, _FF)),
            _full_spec((_FF, _D)), _full_spec((1, _D)),
            _full_spec((_D, 3)), _full_spec((1, 3)),
        ],
        out_specs=pl.BlockSpec((bb, 3), lambda i: (i, 0)),
        out_shape=jax.ShapeDtypeStruct((_B, 3), jnp.float32),
    )(hcls, att2, bp['wo'], bp['bo'].reshape(1, _D),
      bp['ln2_g'].reshape(1, _D), bp['ln2_b'].reshape(1, _D), cent_t,
      bp['w1'], bp['b1'], bp['w2'], bp['b2'],
      bp['ws1'], bp['bs1'].reshape(1, _FF), bp['ws2'],
      bp['bs2'].reshape(1, _D), p['fc_w'], p['fc_b'].reshape(1, 3))


def kernel(x, params):
    p = params
    b0, b1 = p['blocks']
    cent_t = p['centroids'].T

    h0 = _embed(x, p)
    hf = h0.reshape(_T, _D)

    qkv1 = _qkv(hf, b0)
    att1 = _attn1(qkv1.reshape(_B, _S, 3 * _D))
    h1a = _proj(hf, att1.reshape(_T, _D), b0)
    h1 = _moe_dense(h1a, b0, cent_t)

    qkv2 = _qkv(h1, b1)
    att2 = _attn2(qkv2.reshape(_B, _S, 3 * _D))
    hcls = h1.reshape(_B, _S, _D)[:, 0, :]
    return _final(hcls, att2.reshape(_B, _D), b1, cent_t, p)


# speedup vs baseline: 2.0412x; 2.0412x over previous
import math

import jax
import jax.numpy as jnp
from jax import lax
from jax.experimental import pallas as pl
from jax.experimental.pallas import tpu as pltpu

_B, _NF, _D, _FF, _E, _H = 1024, 20, 256, 512, 5, 8
_S = _NF + 1
_T = _B * _S
_DH = _D // _H
_TILE = 512
_NT = _T // _TILE
_G1 = 8
_G2 = 16
_NEG = -1e30


def _f32dot(a, b):
    return jnp.dot(a.astype(jnp.bfloat16), b.astype(jnp.bfloat16),
                   preferred_element_type=jnp.float32)


def _bdg(a, b, dims):
    return lax.dot_general(a.astype(jnp.bfloat16), b.astype(jnp.bfloat16),
                           dims, preferred_element_type=jnp.float32)


def _ln(x, g, b):
    m = jnp.mean(x, axis=-1, keepdims=True)
    v = jnp.mean((x - m) ** 2, axis=-1, keepdims=True)
    return (x - m) / jnp.sqrt(v + 1e-5) * g + b


def _full_spec(shape):
    n = len(shape)
    return pl.BlockSpec(shape, lambda i, _n=n: (0,) * _n)


def _div21(x):
    return (x * 3121) >> 16


def _embed_body(x_ref, w_ref, b_ref, cls_ref, g_ref, bb_ref, o_ref):
    x = x_ref[...]
    w = w_ref[...][None]
    b = b_ref[...][None]
    feat = x[:, :, None] * w + b
    cls = jnp.broadcast_to(cls_ref[...][None], (x.shape[0], 1, _D))
    h = jnp.concatenate([cls, feat], axis=1)
    o_ref[...] = _ln(h, g_ref[...][None], bb_ref[...][None])


def _embed(x, p):
    bb = 256
    return pl.pallas_call(
        _embed_body,
        grid=(_B // bb,),
        in_specs=[
            pl.BlockSpec((bb, _NF), lambda i: (i, 0)),
            _full_spec((1, _D)), _full_spec((1, _D)), _full_spec((1, _D)),
            _full_spec((1, _D)), _full_spec((1, _D)),
        ],
        out_specs=pl.BlockSpec((bb, _S, _D), lambda i: (i, 0, 0)),
        out_shape=jax.ShapeDtypeStruct((_B, _S, _D), jnp.float32),
    )(x, p['con_w'].reshape(1, _D), p['con_b'].reshape(1, _D),
      p['cls'].reshape(1, _D), p['norm_g'].reshape(1, _D),
      p['norm_b'].reshape(1, _D))


def _qkv_body(h_ref, g_ref, b_ref, w_ref, bias_ref, o_ref):
    u = _ln(h_ref[...], g_ref[...], b_ref[...])
    o_ref[...] = _f32dot(u, w_ref[...]) + bias_ref[...]


def _qkv(h, bp):
    return pl.pallas_call(
        _qkv_body,
        grid=(_NT,),
        in_specs=[
            pl.BlockSpec((_TILE, _D), lambda i: (i, 0)),
            _full_spec((1, _D)), _full_spec((1, _D)),
            _full_spec((_D, 3 * _D)), _full_spec((1, 3 * _D)),
        ],
        out_specs=pl.BlockSpec((_TILE, 3 * _D), lambda i: (i, 0)),
        out_shape=jax.ShapeDtypeStruct((_T, 3 * _D), jnp.float32),
    )(h, bp['ln1_g'].reshape(1, _D), bp['ln1_b'].reshape(1, _D),
      bp['wqkv'], bp['bqkv'].reshape(1, 3 * _D))


def _split_heads(t):
    return jnp.concatenate([t[:, :, _DH * h:_DH * (h + 1)] for h in range(_H)],
                           axis=1)


def _attn1_body(qkv_ref, o_ref):
    u = qkv_ref[...]
    qh = _split_heads(u[:, :, :_D])
    kh = _split_heads(u[:, :, _D:2 * _D])
    vh = _split_heads(u[:, :, 2 * _D:])
    s = _bdg(qh, kh, (((2,), (2,)), ((0,), (0,))))
    hs = _H * _S
    rh = _div21(lax.broadcasted_iota(jnp.int32, (hs, hs), 0))
    ch = _div21(lax.broadcasted_iota(jnp.int32, (hs, hs), 1))
    s = jnp.where((rh == ch)[None], s * (1.0 / math.sqrt(_DH)), _NEG)
    s = s - jnp.max(s, axis=-1, keepdims=True)
    e = jnp.exp(s)
    a = e / jnp.sum(e, axis=-1, keepdims=True)
    o = _bdg(a, vh, (((2,), (1,)), ((0,), (0,))))
    o_ref[...] = jnp.concatenate(
        [o[:, _S * h:_S * (h + 1), :] for h in range(_H)], axis=2)


def _attn1(qkv):
    return pl.pallas_call(
        _attn1_body,
        grid=(_B // _G1,),
        in_specs=[pl.BlockSpec((_G1, _S, 3 * _D), lambda i: (i, 0, 0))],
        out_specs=pl.BlockSpec((_G1, _S, _D), lambda i: (i, 0, 0)),
        out_shape=jax.ShapeDtypeStruct((_B, _S, _D), jnp.float32),
    )(qkv)


def _attn2_body(qkv_ref, o_ref):
    u = qkv_ref[...]
    q0 = u[:, 0:1, :_D]
    qh = _split_heads(q0)
    kh = _split_heads(u[:, :, _D:2 * _D])
    vh = _split_heads(u[:, :, 2 * _D:])
    s = _bdg(qh, kh, (((2,), (2,)), ((0,), (0,))))
    hs = _H * _S
    rh = lax.broadcasted_iota(jnp.int32, (_H, hs), 0)
    ch = _div21(lax.broadcasted_iota(jnp.int32, (_H, hs), 1))
    s = jnp.where((rh == ch)[None], s * (1.0 / math.sqrt(_DH)), _NEG)
    s = s - jnp.max(s, axis=-1, keepdims=True)
    e = jnp.exp(s)
    a = e / jnp.sum(e, axis=-1, keepdims=True)
    o = _bdg(a, vh, (((2,), (1,)), ((0,), (0,))))
    o_ref[...] = jnp.concatenate(
        [o[:, h:h + 1, :] for h in range(_H)], axis=2)


def _attn2(qkv):
    return pl.pallas_call(
        _attn2_body,
        grid=(_B // _G2,),
        in_specs=[pl.BlockSpec((_G2, _S, 3 * _D), lambda i: (i, 0, 0))],
        out_specs=pl.BlockSpec((_G2, 1, _D), lambda i: (i, 0, 0)),
        out_shape=jax.ShapeDtypeStruct((_B, 1, _D), jnp.float32),
    )(qkv)


def _proj_body(h_ref, a_ref, w_ref, b_ref, o_ref):
    o_ref[...] = h_ref[...] + _f32dot(a_ref[...], w_ref[...]) + b_ref[...]


def _proj(h, att, bp):
    return pl.pallas_call(
        _proj_body,
        grid=(_NT,),
        in_specs=[
            pl.BlockSpec((_TILE, _D), lambda i: (i, 0)),
            pl.BlockSpec((_TILE, _D), lambda i: (i, 0)),
            _full_spec((_D, _D)), _full_spec((1, _D)),
        ],
        out_specs=pl.BlockSpec((_TILE, _D), lambda i: (i, 0)),
        out_shape=jax.ShapeDtypeStruct((_T, _D), jnp.float32),
    )(h, att, bp['wo'], bp['bo'].reshape(1, _D))


def _moe_math(h, g2, b2, cent, w1, b1, w2, b2e, ws1, bs1, ws2, bs2):
    u2 = _ln(h, g2, b2)
    logits = _f32dot(u2, cent)
    mx = jnp.max(logits, axis=-1, keepdims=True)
    eg = jnp.exp(logits - mx)
    gate = eg / jnp.sum(eg, axis=-1, keepdims=True)
    iot = lax.broadcasted_iota(jnp.int32, logits.shape, 1)
    top = jnp.min(jnp.where(logits == mx, iot, _E), axis=-1, keepdims=True)
    acc = jnp.zeros_like(h)
    for e in range(_E):
        t1 = jax.nn.gelu(_f32dot(u2, w1[e]) + b1[e][None])
        t2 = _f32dot(t1, w2[e]) + b2e[e][None]
        sel = jnp.where(top == e, gate[:, e:e + 1], 0.0)
        acc = acc + sel * t2
    ys = _f32dot(jax.nn.gelu(_f32dot(u2, ws1) + bs1), ws2) + bs2
    return h + acc + ys


def _moe_body(h_ref, g2_ref, b2_ref, cent_ref, w1_ref, b1_ref, w2_ref,
              b2e_ref, ws1_ref, bs1_ref, ws2_ref, bs2_ref, o_ref):
    o_ref[...] = _moe_math(
        h_ref[...], g2_ref[...], b2_ref[...], cent_ref[...], w1_ref[...],
        b1_ref[...], w2_ref[...], b2e_ref[...], ws1_ref[...], bs1_ref[...],
        ws2_ref[...], bs2_ref[...])


def _moe_dense(h, bp, cent_t):
    return pl.pallas_call(
        _moe_body,
        grid=(_NT,),
        in_specs=[
            pl.BlockSpec((_TILE, _D), lambda i: (i, 0)),
            _full_spec((1, _D)), _full_spec((1, _D)), _full_spec((_D, _E)),
            _full_spec((_E, _D, _FF)), _full_spec((_E, _FF)),
            _full_spec((_E, _FF, _D)), _full_spec((_E, _D)),
            _full_spec((_D, _FF)), _full_spec((1, _FF)),
            _full_spec((_FF, _D)), _full_spec((1, _D)),
        ],
        out_specs=pl.BlockSpec((_TILE, _D), lambda i: (i, 0)),
        out_shape=jax.ShapeDtypeStruct((_T, _D), jnp.float32),
    )(h, bp['ln2_g'].reshape(1, _D), bp['ln2_b'].reshape(1, _D), cent_t,
      bp['w1'], bp['b1'], bp['w2'], bp['b2'],
      bp['ws1'], bp['bs1'].reshape(1, _FF), bp['ws2'],
      bp['bs2'].reshape(1, _D))


def _final_body(hc_ref, a2_ref, wo_ref, bo_ref, g2_ref, b2_ref, cent_ref,
                w1_ref, b1_ref, w2_ref, b2e_ref, ws1_ref, bs1_ref, ws2_ref,
                bs2_ref, fcw_ref, fcb_ref, o_ref):
    c = hc_ref[...] + _f32dot(a2_ref[...], wo_ref[...]) + bo_ref[...]
    o = _moe_math(c, g2_ref[...], b2_ref[...], cent_ref[...], w1_ref[...],
                  b1_ref[...], w2_ref[...], b2e_ref[...], ws1_ref[...],
                  bs1_ref[...], ws2_ref[...], bs2_ref[...])
    o_ref[...] = _f32dot(o, fcw_ref[...]) + fcb_ref[...]


def _final(hcls, att2, bp, cent_t, p):
    bb = 512
    return pl.pallas_call(
        _final_body,
        grid=(_B // bb,),
        in_specs=[
            pl.BlockSpec((bb, _D), lambda i: (i, 0)),
            pl.BlockSpec((bb, _D), lambda i: (i, 0)),
            _full_spec((_D, _D)), _full_spec((1, _D)),
            _full_spec((1, _D)), _full_spec((1, _D)), _full_spec((_D, _E)),
            _full_spec((_E, _D, _FF)), _full_spec((_E, _FF)),
            _full_spec((_E, _FF, _D)), _full_spec((_E, _D)),
            _full_spec((_D, _FF)), _full_spec((1, _FF)),
            _full_spec((_FF, _D)), _full_spec((1, _D)),
            _full_spec((_D, 3)), _full_spec((1, 3)),
        ],
        out_specs=pl.BlockSpec((bb, 3), lambda i: (i, 0)),
        out_shape=jax.ShapeDtypeStruct((_B, 3), jnp.float32),
    )(hcls, att2, bp['wo'], bp['bo'].reshape(1, _D),
      bp['ln2_g'].reshape(1, _D), bp['ln2_b'].reshape(1, _D), cent_t,
      bp['w1'], bp['b1'], bp['w2'], bp['b2'],
      bp['ws1'], bp['bs1'].reshape(1, _FF), bp['ws2'],
      bp['bs2'].reshape(1, _D), p['fc_w'], p['fc_b'].reshape(1, 3))


def kernel(x, params):
    p = params
    b0, b1 = p['blocks']
    cent_t = p['centroids'].T

    h0 = _embed(x, p)
    hf = h0.reshape(_T, _D)

    qkv1 = _qkv(hf, b0)
    att1 = _attn1(qkv1.reshape(_B, _S, 3 * _D))
    h1a = _proj(hf, att1.reshape(_T, _D), b0)
    h1 = _moe_dense(h1a, b0, cent_t)

    qkv2 = _qkv(h1, b1)
    att2 = _attn2(qkv2.reshape(_B, _S, 3 * _D))
    hcls = h1.reshape(_B, _S, _D)[:, 0, :]
    return _final(hcls, att2.reshape(_B, _D), b1, cent_t, p)
